# baseline (speedup 1.0000x reference)
import jax
import jax.numpy as jnp
from jax import lax
from jax.experimental import pallas as pl
from jax.experimental.pallas import tpu as pltpu
from jax.experimental.pallas import tpu_sc as plsc

VOCAB = 100000
SEQ = 2048
DIM = 128
BATCH = 16

NC = 2
NS = 16
NW = NC * NS
LANES = 16
VPR = DIM // LANES

PW = SEQ // NW
STEPS = BATCH
RPS = PW
NBUF = 6
GAHEAD = 3


def _add_pos(rows_v, pos_v, k):
    @plsc.parallel_loop(0, RPS, step=1, unroll=4)
    def _(r):
        for u in range(VPR):
            off = u * LANES
            x = pos_v[r, pl.ds(off, LANES)]
            plsc.addupdate(rows_v.at[k, r, pl.ds(off, LANES)], x)


def _body(vals_hbm, tok_hbm, pos_hbm, out_hbm, idx_v, pos_v, rows_v,
          gsem, ssem):
    cid = lax.axis_index("c")
    sid = lax.axis_index("s")
    wid = sid * NC + cid
    p0 = wid * PW

    idx_cps = [
        pltpu.async_copy(vals_hbm.at[pl.ds(b * SEQ + p0, PW)],
                         idx_v.at[b], gsem)
        for b in range(BATCH)
    ]
    pltpu.sync_copy(pos_hbm.at[pl.ds(p0, PW)], pos_v)
    for cp in idx_cps:
        cp.wait()

    gathers = [None] * STEPS
    stores = [None] * STEPS

    def start_gather(s):
        gathers[s] = pltpu.async_copy(
            tok_hbm.at[idx_v.at[s]], rows_v.at[s % NBUF], gsem)

    def start_store(s):
        stores[s] = pltpu.async_copy(
            rows_v.at[s % NBUF], out_hbm.at[pl.ds(s * SEQ + p0, PW)], ssem)

    for s in range(GAHEAD):
        start_gather(s)

    for s in range(STEPS):
        k = s % NBUF
        gathers[s].wait()
        j = s + GAHEAD
        if j < STEPS:
            if j >= NBUF:
                stores[j - NBUF].wait()
            start_gather(j)
        _add_pos(rows_v, pos_v, k)
        start_store(s)

    for s in range(STEPS - NBUF, STEPS):
        stores[s].wait()


@jax.jit
def kernel(values, tok_table, pos_table):
    vals = values.reshape(BATCH * SEQ).astype(jnp.int32)
    mesh = plsc.VectorSubcoreMesh(core_axis_name="c", subcore_axis_name="s")
    out = pl.kernel(
        _body,
        out_type=jax.ShapeDtypeStruct((BATCH * SEQ, DIM), jnp.float32),
        mesh=mesh,
        scratch_types=[
            pltpu.VMEM((BATCH, PW), jnp.int32),
            pltpu.VMEM((PW, DIM), jnp.float32),
            pltpu.VMEM((NBUF, RPS, DIM), jnp.float32),
            pltpu.SemaphoreType.DMA,
            pltpu.SemaphoreType.DMA,
        ],
    )(vals, tok_table, pos_table)
    return out.reshape(BATCH, SEQ, DIM)
